# baseline (speedup 1.0000x reference)
import functools

import jax
import jax.numpy as jnp
import numpy as np
from jax.experimental import pallas as pl
from jax.experimental.pallas import tpu as pltpu


def _fps_body(x_ref, y_ref, z_ref, ox_ref, oy_ref, oz_ref, npoint):
    x = x_ref[...]
    y = y_ref[...]
    z = z_ref[...]
    b, n = x.shape
    jn = jax.lax.broadcasted_iota(jnp.int32, (b, n), 1)
    js = jax.lax.broadcasted_iota(jnp.int32, (b, npoint), 1)

    def body(i, st):
        dist, far, ax, ay, az = st
        sel = jn == far[:, None]
        cx = jnp.sum(jnp.where(sel, x, 0.0), axis=1)
        cy = jnp.sum(jnp.where(sel, y, 0.0), axis=1)
        cz = jnp.sum(jnp.where(sel, z, 0.0), axis=1)
        here = js == i
        ax = jnp.where(here, cx[:, None], ax)
        ay = jnp.where(here, cy[:, None], ay)
        az = jnp.where(here, cz[:, None], az)
        dx = x - cx[:, None]
        dy = y - cy[:, None]
        dz = z - cz[:, None]
        d = (dx * dx + dy * dy) + dz * dz
        dist = jnp.minimum(dist, d)
        m = jnp.max(dist, axis=1)
        far = jnp.min(jnp.where(dist == m[:, None], jn, n), axis=1)
        return dist, far, ax, ay, az

    dist0 = jnp.full((b, n), 1e10, jnp.float32)
    far0 = jnp.zeros((b,), jnp.int32)
    z0 = jnp.zeros((b, npoint), jnp.float32)
    _, _, ax, ay, az = jax.lax.fori_loop(
        0, npoint, body, (dist0, far0, z0, z0, z0))
    ox_ref[...] = ax
    oy_ref[...] = ay
    oz_ref[...] = az


def _fps_coords(xyz, npoint):
    B, N, _ = xyz.shape
    x = xyz[:, :, 0]
    y = xyz[:, :, 1]
    z = xyz[:, :, 2]
    outs = pl.pallas_call(
        functools.partial(_fps_body, npoint=npoint),
        out_shape=[jax.ShapeDtypeStruct((B, npoint), jnp.float32)] * 3,
    )(x, y, z)
    return jnp.stack(outs, axis=-1)


def _ballq_body(q_ref, p_ref, o_ref, *, r2, K, nb):
    q = q_ref[0]
    p = p_ref[0]
    SC = q.shape[0]
    n = p.shape[0]
    qn = jnp.sum(q * q, axis=-1)
    pn = jnp.sum(p * p, axis=-1)
    dot = jax.lax.dot_general(q, p, (((1,), (1,)), ((), ())),
                              preferred_element_type=jnp.float32)
    d = qn[:, None] + pn[None, :] - 2.0 * dot
    mask = jnp.where(d <= r2, 1.0, 0.0)

    mr = mask.reshape(SC * nb, 128)
    li = jax.lax.broadcasted_iota(jnp.int32, (128, 128), 0)
    lj = jax.lax.broadcasted_iota(jnp.int32, (128, 128), 1)
    tri = jnp.where(li <= lj, 1.0, 0.0)
    within = jax.lax.dot_general(mr, tri, (((1,), (0,)), ((), ())),
                                 preferred_element_type=jnp.float32)
    bs = within[:, 127].reshape(SC, nb)
    bi = jax.lax.broadcasted_iota(jnp.int32, (nb, nb), 0)
    bj = jax.lax.broadcasted_iota(jnp.int32, (nb, nb), 1)
    utri = jnp.where(bi < bj, 1.0, 0.0)
    excl = jax.lax.dot_general(bs, utri, (((1,), (0,)), ((), ())),
                               preferred_element_type=jnp.float32)
    rank = within.reshape(SC, nb, 128) + excl[:, :, None]
    cnt = excl[:, nb - 1] + bs[:, nb - 1]

    blk = jax.lax.broadcasted_iota(jnp.int32, (SC, nb, 128), 1)
    lane = jax.lax.broadcasted_iota(jnp.int32, (SC, nb, 128), 2)
    jm = (blk * 128 + lane).astype(jnp.float32) * mask.reshape(SC, nb, 128)

    acc = jnp.zeros((SC, K), jnp.float32)
    ks = jax.lax.broadcasted_iota(jnp.int32, (SC, K), 1).astype(jnp.float32)
    for r in range(K):
        sel = jnp.sum(jnp.where(rank == float(r + 1), jm, 0.0), axis=(1, 2))
        acc = acc + jnp.where(ks == float(r), sel[:, None], 0.0)
    first = jnp.where(cnt > 0.0, acc[:, 0], float(n - 1))
    out = jnp.where(ks < cnt[:, None], acc, first[:, None])
    o_ref[0] = out.astype(jnp.int32)


def _ball_query_pl(radius, K, xyz, new_xyz):
    B, S, _ = new_xyz.shape
    n = xyz.shape[1]
    SC = min(S, 64)
    nb = n // 128
    body = functools.partial(_ballq_body, r2=radius * radius, K=K, nb=nb)
    return pl.pallas_call(
        body,
        grid=(B, S // SC),
        in_specs=[
            pl.BlockSpec((1, SC, 3), lambda b, i: (b, i, 0)),
            pl.BlockSpec((1, n, 3), lambda b, i: (b, 0, 0)),
        ],
        out_specs=pl.BlockSpec((1, SC, K), lambda b, i: (b, i, 0)),
        out_shape=jax.ShapeDtypeStruct((B, S, K), jnp.int32),
    )(new_xyz, xyz)


def _knn_body(q_ref, p_ref, oi_ref, ow_ref, *, K, want_w):
    q = q_ref[0]
    p = p_ref[0]
    SC = q.shape[0]
    n = p.shape[0]
    qn = jnp.sum(q * q, axis=-1)
    pn = jnp.sum(p * p, axis=-1)
    dot = jax.lax.dot_general(q, p, (((1,), (1,)), ((), ())),
                              preferred_element_type=jnp.float32)
    d = qn[:, None] + pn[None, :] - 2.0 * dot
    jn = jax.lax.broadcasted_iota(jnp.int32, (SC, n), 1)
    ks = jax.lax.broadcasted_iota(jnp.int32, (SC, K), 1)
    acci = jnp.zeros((SC, K), jnp.int32)
    accd = jnp.zeros((SC, K), jnp.float32)
    work = d
    for t in range(K):
        mn = jnp.min(work, axis=1)
        pick = jnp.min(jnp.where(work == mn[:, None], jn, n), axis=1)
        here = ks == t
        acci = jnp.where(here, pick[:, None], acci)
        accd = jnp.where(here, mn[:, None], accd)
        work = jnp.where(jn == pick[:, None], jnp.inf, work)
    oi_ref[0] = acci
    if want_w:
        d3 = jnp.maximum(accd, 0.0)
        w = 1.0 / (d3 + 1e-8)
        ow_ref[0] = w / jnp.sum(w, axis=-1, keepdims=True)


def _knn_pl(K, xyz_q, xyz_p, want_w):
    B, S, _ = xyz_q.shape
    n = xyz_p.shape[1]
    SC = min(S, 256)
    body = functools.partial(_knn_body, K=K, want_w=want_w)
    out_shape = [jax.ShapeDtypeStruct((B, S, K), jnp.int32),
                 jax.ShapeDtypeStruct((B, S, K), jnp.float32)]
    idx, w = pl.pallas_call(
        body,
        grid=(B, S // SC),
        in_specs=[
            pl.BlockSpec((1, SC, 3), lambda b, i: (b, i, 0)),
            pl.BlockSpec((1, n, 3), lambda b, i: (b, 0, 0)),
        ],
        out_specs=[pl.BlockSpec((1, SC, K), lambda b, i: (b, i, 0)),
                   pl.BlockSpec((1, SC, K), lambda b, i: (b, i, 0))],
        out_shape=out_shape,
    )(xyz_q, xyz_p)
    return (idx, w) if want_w else (idx, None)


def _stats_rows(z, st_ref, step):
    s = jnp.sum(z, axis=0)
    r8 = jax.lax.broadcasted_iota(jnp.int32, (8, z.shape[1]), 0)
    rows = jnp.where(r8 == 0, s[None, :], 0.0)

    @pl.when(step == 0)
    def _():
        st_ref[...] = jnp.zeros_like(st_ref)

    st_ref[...] = st_ref[...] + rows


def _var_body(z_ref, st_ref, sto_ref, *, M):
    z = z_ref[...]
    m = st_ref[0] / M
    zc = z - m[None, :]
    v = jnp.sum(zc * zc, axis=0)
    r8 = jax.lax.broadcasted_iota(jnp.int32, (8, z.shape[1]), 0)
    rows = jnp.where(r8 == 1, v[None, :], 0.0)

    @pl.when(pl.program_id(0) == 0)
    def _():
        sto_ref[...] = st_ref[...]

    sto_ref[...] = sto_ref[...] + rows


def _var_pass(z, st):
    M, C = z.shape
    CH = min(M, 4096)
    body = functools.partial(_var_body, M=float(M))
    return pl.pallas_call(
        body,
        grid=(M // CH,),
        in_specs=[
            pl.BlockSpec((CH, C), lambda i: (i, 0)),
            pl.BlockSpec((8, C), lambda i: (0, 0)),
        ],
        out_specs=pl.BlockSpec((8, C), lambda i: (0, 0)),
        out_shape=jax.ShapeDtypeStruct((8, C), jnp.float32),
    )(z, st)


def _bn_fold(z, st_ref, g_ref, be_ref, M):
    m = st_ref[0] / M
    v = st_ref[1] / M
    sc = g_ref[...] / jnp.sqrt(v + 1e-5)
    sh = be_ref[...] - m * sc
    return jnp.maximum(z * sc[None, :] + sh[None, :], 0.0)


def _mm_stats_body(x_ref, w_ref, b_ref, z_ref, st_ref):
    z = jax.lax.dot_general(x_ref[...], w_ref[...], (((1,), (1,)), ((), ())),
                            preferred_element_type=jnp.float32) + b_ref[...]
    z_ref[...] = z
    _stats_rows(z, st_ref, pl.program_id(0))


def _mm_stats(x, W, b):
    M, Cin = x.shape
    if Cin % 128 != 0:
        pad = 128 - Cin % 128
        x = jnp.concatenate([x, jnp.zeros((M, pad), jnp.float32)], axis=1)
        W = jnp.concatenate(
            [W, jnp.zeros((W.shape[0], pad), jnp.float32)], axis=1)
        Cin = Cin + pad
    Cout = W.shape[0]
    CH = min(M, 4096)
    z, st = pl.pallas_call(
        _mm_stats_body,
        grid=(M // CH,),
        in_specs=[
            pl.BlockSpec((CH, Cin), lambda i: (i, 0)),
            pl.BlockSpec((Cout, Cin), lambda i: (0, 0)),
            pl.BlockSpec((Cout,), lambda i: (0,)),
        ],
        out_specs=[pl.BlockSpec((CH, Cout), lambda i: (i, 0)),
                   pl.BlockSpec((8, Cout), lambda i: (0, 0))],
        out_shape=[jax.ShapeDtypeStruct((M, Cout), jnp.float32),
                   jax.ShapeDtypeStruct((8, Cout), jnp.float32)],
    )(x, W, b)
    return z, _var_pass(z, st)


def _mm2_body(x3_ref, xf_ref, w3_ref, wf_ref, b_ref, z_ref, st_ref):
    p3 = jax.lax.dot_general(x3_ref[...], w3_ref[...],
                             (((1,), (1,)), ((), ())),
                             preferred_element_type=jnp.float32)
    pf = jax.lax.dot_general(xf_ref[...], wf_ref[...],
                             (((1,), (1,)), ((), ())),
                             preferred_element_type=jnp.float32)
    z = (p3 + pf) + b_ref[...]
    z_ref[...] = z
    _stats_rows(z, st_ref, pl.program_id(0))


def _mm2_stats(x3, xf, W, b):
    M, Cf = xf.shape
    Cout = W.shape[0]
    W3 = W[:, :3]
    Wf = W[:, 3:]
    CH = min(M, 4096)
    z, st = pl.pallas_call(
        _mm2_body,
        grid=(M // CH,),
        in_specs=[
            pl.BlockSpec((CH, 3), lambda i: (i, 0)),
            pl.BlockSpec((CH, Cf), lambda i: (i, 0)),
            pl.BlockSpec((Cout, 3), lambda i: (0, 0)),
            pl.BlockSpec((Cout, Cf), lambda i: (0, 0)),
            pl.BlockSpec((Cout,), lambda i: (0,)),
        ],
        out_specs=[pl.BlockSpec((CH, Cout), lambda i: (i, 0)),
                   pl.BlockSpec((8, Cout), lambda i: (0, 0))],
        out_shape=[jax.ShapeDtypeStruct((M, Cout), jnp.float32),
                   jax.ShapeDtypeStruct((8, Cout), jnp.float32)],
    )(x3, xf, W3, Wf, b)
    return z, _var_pass(z, st)


def _bn_mm_body(z_ref, st_ref, g_ref, be_ref, w_ref, b_ref,
                zo_ref, sto_ref, *, M):
    y = _bn_fold(z_ref[...], st_ref, g_ref, be_ref, M)
    z2 = jax.lax.dot_general(y, w_ref[...], (((1,), (1,)), ((), ())),
                             preferred_element_type=jnp.float32) + b_ref[...]
    zo_ref[...] = z2
    _stats_rows(z2, sto_ref, pl.program_id(0))


def _bn_mm(z, st, g, be, W, b):
    M, Cin = z.shape
    Cout = W.shape[0]
    CH = min(M, 4096)
    body = functools.partial(_bn_mm_body, M=float(M))
    z2, st2 = pl.pallas_call(
        body,
        grid=(M // CH,),
        in_specs=[
            pl.BlockSpec((CH, Cin), lambda i: (i, 0)),
            pl.BlockSpec((8, Cin), lambda i: (0, 0)),
            pl.BlockSpec((Cin,), lambda i: (0,)),
            pl.BlockSpec((Cin,), lambda i: (0,)),
            pl.BlockSpec((Cout, Cin), lambda i: (0, 0)),
            pl.BlockSpec((Cout,), lambda i: (0,)),
        ],
        out_specs=[pl.BlockSpec((CH, Cout), lambda i: (i, 0)),
                   pl.BlockSpec((8, Cout), lambda i: (0, 0))],
        out_shape=[jax.ShapeDtypeStruct((M, Cout), jnp.float32),
                   jax.ShapeDtypeStruct((8, Cout), jnp.float32)],
    )(z, st, g, be, W, b)
    return z2, _var_pass(z2, st2)


def _sa_k1_body(zg_ref, gx_ref, wx_ref, b_ref, zo_ref, sto_ref):
    z = zg_ref[...] + jax.lax.dot_general(
        gx_ref[...], wx_ref[...], (((1,), (1,)), ((), ())),
        preferred_element_type=jnp.float32) + b_ref[...]
    zo_ref[...] = z
    _stats_rows(z, sto_ref, pl.program_id(0))


def _sa_k1(zg, gxrel, Wxyz, b):
    M, C = zg.shape
    CH = min(M, 4096)
    z, st = pl.pallas_call(
        _sa_k1_body,
        grid=(M // CH,),
        in_specs=[
            pl.BlockSpec((CH, C), lambda i: (i, 0)),
            pl.BlockSpec((CH, 3), lambda i: (i, 0)),
            pl.BlockSpec((C, 3), lambda i: (0, 0)),
            pl.BlockSpec((C,), lambda i: (0,)),
        ],
        out_specs=[pl.BlockSpec((CH, C), lambda i: (i, 0)),
                   pl.BlockSpec((8, C), lambda i: (0, 0))],
        out_shape=[jax.ShapeDtypeStruct((M, C), jnp.float32),
                   jax.ShapeDtypeStruct((8, C), jnp.float32)],
    )(zg, gxrel, Wxyz, b)
    return z, st


def _bn_pool_body(z_ref, st_ref, g_ref, be_ref, o_ref, *, M, K):
    y = _bn_fold(z_ref[...], st_ref, g_ref, be_ref, M)
    q, c = o_ref.shape
    o_ref[...] = jnp.max(y.reshape(q, K, c), axis=1)


def _bn_pool(z, st, g, be, Q, K, CHq):
    M, C = z.shape
    body = functools.partial(_bn_pool_body, M=float(M), K=K)
    return pl.pallas_call(
        body,
        grid=(Q // CHq,),
        in_specs=[
            pl.BlockSpec((CHq * K, C), lambda i: (i, 0)),
            pl.BlockSpec((8, C), lambda i: (0, 0)),
            pl.BlockSpec((C,), lambda i: (0,)),
            pl.BlockSpec((C,), lambda i: (0,)),
        ],
        out_specs=pl.BlockSpec((CHq, C), lambda i: (i, 0)),
        out_shape=jax.ShapeDtypeStruct((Q, C), jnp.float32),
    )(z, st, g, be)


def _boundary_body(f_ref, n_ref, w_ref, b_ref, o_ref):
    f = f_ref[...]
    q, c = f.shape
    nb = n_ref[...].reshape(q, 8, c)
    edge = jnp.max(jnp.abs(nb - f[:, None, :]), axis=1)
    zg = jax.lax.dot_general(edge, w_ref[...], (((1,), (1,)), ((), ())),
                             preferred_element_type=jnp.float32) + b_ref[...]
    gate = jax.nn.sigmoid(zg)
    o_ref[...] = f * (1.0 + gate)


def _boundary_gate(feat, neigh, W, b):
    Q, C = feat.shape
    CHq = min(Q, 256)
    return pl.pallas_call(
        _boundary_body,
        grid=(Q // CHq,),
        in_specs=[
            pl.BlockSpec((CHq, C), lambda i: (i, 0)),
            pl.BlockSpec((CHq * 8, C), lambda i: (i, 0)),
            pl.BlockSpec((C, C), lambda i: (0, 0)),
            pl.BlockSpec((C,), lambda i: (0,)),
        ],
        out_specs=pl.BlockSpec((CHq, C), lambda i: (i, 0)),
        out_shape=jax.ShapeDtypeStruct((Q, C), jnp.float32),
    )(feat, neigh, W, b)


def _fp_k1_body(g_ref, w3_ref, f1_ref, wa_ref, b_ref, zo_ref, sto_ref,
                *, has_f1):
    w3 = w3_ref[...]
    q = w3.shape[0]
    c = zo_ref.shape[1]
    g3 = g_ref[...].reshape(q, 3, c)
    interp = (g3[:, 0, :] * w3[:, 0:1] + g3[:, 1, :] * w3[:, 1:2]
              + g3[:, 2, :] * w3[:, 2:3])
    z = interp + b_ref[...]
    if has_f1:
        z = z + jax.lax.dot_general(
            f1_ref[...], wa_ref[...], (((1,), (1,)), ((), ())),
            preferred_element_type=jnp.float32)
    zo_ref[...] = z
    _stats_rows(z, sto_ref, pl.program_id(0))


def _fp_k1(G, w, feat1, Wa, b):
    Q3, C = G.shape
    Q = Q3 // 3
    CHq = min(Q, 1024)
    has_f1 = feat1 is not None
    if not has_f1:
        feat1 = jnp.zeros((Q, 8), jnp.float32)
        Wa = jnp.zeros((C, 8), jnp.float32)
    Cf = feat1.shape[1]
    body = functools.partial(_fp_k1_body, has_f1=has_f1)
    z, st = pl.pallas_call(
        body,
        grid=(Q // CHq,),
        in_specs=[
            pl.BlockSpec((CHq * 3, C), lambda i: (i, 0)),
            pl.BlockSpec((CHq, 3), lambda i: (i, 0)),
            pl.BlockSpec((CHq, Cf), lambda i: (i, 0)),
            pl.BlockSpec((C, Cf), lambda i: (0, 0)),
            pl.BlockSpec((C,), lambda i: (0,)),
        ],
        out_specs=[pl.BlockSpec((CHq, C), lambda i: (i, 0)),
                   pl.BlockSpec((8, C), lambda i: (0, 0))],
        out_shape=[jax.ShapeDtypeStruct((Q, C), jnp.float32),
                   jax.ShapeDtypeStruct((8, C), jnp.float32)],
    )(G, w, feat1, Wa, b)
    return z, st


def _head_out_body(z_ref, st_ref, g_ref, be_ref, w2_ref, b2_ref, o_ref,
                   *, M):
    y = _bn_fold(z_ref[...], st_ref, g_ref, be_ref, M)
    o = jax.lax.dot_general(w2_ref[...], y, (((1,), (1,)), ((), ())),
                            preferred_element_type=jnp.float32)
    o_ref[0] = o + b2_ref[...][:, None]


def _head_out(z, st, g, be, W2, b2, B, N):
    M, C = z.shape
    CH = 2048
    nc = N // CH
    body = functools.partial(_head_out_body, M=float(M))
    return pl.pallas_call(
        body,
        grid=(B, nc),
        in_specs=[
            pl.BlockSpec((CH, C), lambda b, i: (b * nc + i, 0)),
            pl.BlockSpec((8, C), lambda b, i: (0, 0)),
            pl.BlockSpec((C,), lambda b, i: (0,)),
            pl.BlockSpec((C,), lambda b, i: (0,)),
            pl.BlockSpec((8, C), lambda b, i: (0, 0)),
            pl.BlockSpec((8,), lambda b, i: (0,)),
        ],
        out_specs=pl.BlockSpec((1, 8, CH), lambda b, i: (b, 0, i)),
        out_shape=jax.ShapeDtypeStruct((B, 8, N), jnp.float32),
    )(z, st, g, be, W2, b2)


def _index_points(p, idx):
    return jax.vmap(lambda pp, ii: pp[ii])(p, idx)


def _sc_gather_flat(table, idx):
    from jax.experimental.pallas import tpu_sc as plsc
    R_, D = table.shape
    M = idx.shape[0]
    info = plsc.get_sparse_core_info()
    NC, NS = info.num_cores, info.num_subcores
    NW = NC * NS
    m_w = M // NW
    chunk = min(128, m_w)
    nch = m_w // chunk
    mesh = plsc.VectorSubcoreMesh(core_axis_name="c", subcore_axis_name="s")

    @functools.partial(
        pl.kernel, mesh=mesh,
        out_type=jax.ShapeDtypeStruct((M, D), jnp.float32),
        scratch_types=[
            pltpu.VMEM((chunk,), jnp.int32),
            pltpu.VMEM((chunk, D), jnp.float32),
            pltpu.SemaphoreType.DMA,
        ],
    )
    def k(table_hbm, idx_hbm, out_hbm, idx_v, rows_v, sem):
        wid = jax.lax.axis_index("s") * NC + jax.lax.axis_index("c")
        base = wid * m_w
        for c in range(nch):
            off = base + c * chunk
            pltpu.sync_copy(idx_hbm.at[pl.ds(off, chunk)], idx_v)
            pltpu.async_copy(table_hbm.at[idx_v], rows_v, sem).wait()
            pltpu.sync_copy(rows_v, out_hbm.at[pl.ds(off, chunk)])

    return k(table, idx)


def _sc_index_points(p, idx):
    B, n, D = p.shape
    flat = (idx.astype(jnp.int32)
            + (jnp.arange(B, dtype=jnp.int32) * n).reshape(
                (B,) + (1,) * (idx.ndim - 1))).reshape(-1)
    out = _sc_gather_flat(p.reshape(B * n, D), flat)
    return out.reshape(idx.shape + (D,))


def _conv_bn_relu(x, l):
    x = x @ l["W"].T + l["b"]
    ax = tuple(range(x.ndim - 1))
    m = jnp.mean(x, axis=ax, keepdims=True)
    v = jnp.var(x, axis=ax, keepdims=True)
    x = (x - m) / jnp.sqrt(v + 1e-5) * l["g"] + l["be"]
    return jax.nn.relu(x)


def _set_abstraction(xyz, feat, npoint, radius, k, layers):
    new_xyz = _fps_coords(xyz, npoint)
    gidx = _ball_query_pl(radius, k, xyz, new_xyz)
    gxyz = _index_points(xyz, gidx) - new_xyz[:, :, None, :]
    gfeat = _index_points(feat, gidx)
    x = jnp.concatenate([gxyz, gfeat], axis=-1)
    for l in layers:
        x = _conv_bn_relu(x, l)
    return new_xyz, jnp.max(x, axis=2)


def _boundary(feat, xyz, p):
    idx, _ = _knn_pl(8, xyz, xyz, want_w=False)
    neigh = _sc_index_points(feat, idx)
    diff = neigh - feat[:, :, None, :]
    edge = jnp.max(jnp.abs(diff), axis=2)
    gate = jax.nn.sigmoid(edge @ p["W"].T + p["b"])
    return feat * (1.0 + gate)


def _feature_prop(xyz1, xyz2, feat1, feat2, layers):
    idx, w = _knn_pl(3, xyz1, xyz2, want_w=True)
    interp = jnp.sum(_sc_index_points(feat2, idx) * w[..., None], axis=2)
    x = interp if feat1 is None else jnp.concatenate([feat1, interp], axis=-1)
    for l in layers:
        x = _conv_bn_relu(x, l)
    return x


def _head_kernel(h_ref, w_ref, b_ref, o_ref):
    h = h_ref[...]
    w = w_ref[...]
    o_ref[...] = jax.lax.dot_general(
        h, w, (((1,), (1,)), ((), ())),
        preferred_element_type=jnp.float32) + b_ref[...]


def _head_project(h, W2, b2):
    B, N, C = h.shape
    M = B * N
    h2 = h.reshape(M, C)
    CH = 8192
    out = pl.pallas_call(
        _head_kernel,
        grid=(M // CH,),
        in_specs=[
            pl.BlockSpec((CH, C), lambda i: (i, 0)),
            pl.BlockSpec((8, C), lambda i: (0, 0)),
            pl.BlockSpec((8,), lambda i: (0,)),
        ],
        out_specs=pl.BlockSpec((CH, 8), lambda i: (i, 0)),
        out_shape=jax.ShapeDtypeStruct((M, 8), jnp.float32),
    )(h2, W2, b2)
    return out.reshape(B, N, 8)


def kernel(xyz, points, params):
    pe = xyz @ params["pe"]["W"].T + params["pe"]["b"]
    f0 = jnp.concatenate([points, pe], axis=-1)
    l1_xyz, l1 = _set_abstraction(xyz, f0, 1024, 0.1, 32, params["sa1"])
    l1 = _boundary(l1, l1_xyz, params["bd1"])
    l2_xyz, l2 = _set_abstraction(l1_xyz, l1, 256, 0.2, 32, params["sa2"])
    l2 = _boundary(l2, l2_xyz, params["bd2"])
    l3_xyz, l3 = _set_abstraction(l2_xyz, l2, 64, 0.4, 32, params["sa3"])
    l3 = _boundary(l3, l3_xyz, params["bd3"])
    l2 = _feature_prop(l2_xyz, l3_xyz, l2, l3, params["fp3"])
    l1 = _feature_prop(l1_xyz, l2_xyz, l1, l2, params["fp2"])
    l0 = _feature_prop(xyz, l1_xyz, None, l1, params["fp1"])
    B, N = l0.shape[0], l0.shape[1]
    hd = params["head"]
    zhd, shd = _mm_stats(l0.reshape(B * N, 128), hd["l1"]["W"], hd["l1"]["b"])
    return _head_out(zhd, shd, hd["l1"]["g"], hd["l1"]["be"],
                     hd["W2"], hd["b2"], B, N)
